# baseline (speedup 1.0000x reference)
import functools

import jax
import jax.numpy as jnp
from jax import lax
from jax.experimental import pallas as pl
from jax.experimental.pallas import tpu as pltpu
from jax.experimental.pallas import tpu_sc as plsc

N_NODES_K = 10000
D = 128
N_EDGES_K = 320000

NC = 2
NS = 16
NW = NC * NS

CHUNK = 64
N_CHUNKS = N_EDGES_K // CHUNK
BASE_CHUNKS = N_CHUNKS // NW
EXTRA_TILES = N_CHUNKS - BASE_CHUNKS * NW

NB = 4
NI = 12
GROUPS = BASE_CHUNKS // NI

ZSTRIPE = N_NODES_K // NS // 16 * 16
LAST_EXTRA = N_NODES_K - NS * ZSTRIPE


def _sc_aggregate(features, eidx):
    mesh = plsc.VectorSubcoreMesh(core_axis_name="c", subcore_axis_name="s")

    @functools.partial(
        pl.kernel,
        out_type=jax.ShapeDtypeStruct((NC, N_NODES_K, D), jnp.float32),
        mesh=mesh,
        scratch_types=(
            [pltpu.VMEM((CHUNK, D), jnp.float32)] * NB
            + [pltpu.VMEM((2 * NI, CHUNK), jnp.int32)]
            + [pltpu.VMEM_SHARED((N_NODES_K, D), jnp.float32)]
            + [pltpu.SemaphoreType.DMA] * (2 * NB + NI + 1)
        ),
    )
    def agg_kernel(feat_hbm, eidx_hbm, out_hbm, *rest):
        bufs = rest[:NB]
        ibuf = rest[NB]
        agg_sh = rest[NB + 1]
        sems = rest[NB + 2:]
        gsems = sems[:NB]
        ssems = sems[NB:2 * NB]
        isems = sems[2 * NB:2 * NB + NI]
        zsem = sems[2 * NB + NI]

        c = lax.axis_index("c")
        s = lax.axis_index("s")
        wid = c * NS + s
        start = wid * BASE_CHUNKS + jnp.minimum(wid, EXTRA_TILES)
        extra = wid < EXTRA_TILES

        zvec = jnp.zeros((16,), jnp.float32)
        for r in range(16):
            for col in range(D // 16):
                bufs[0][r, pl.ds(col * 16, 16)] = zvec

        def zfire(i, carry):
            pltpu.async_copy(bufs[0].at[pl.ds(0, 16)],
                             agg_sh.at[pl.ds(s * ZSTRIPE + i * 16, 16)], zsem)
            return carry
        lax.fori_loop(0, ZSTRIPE // 16, zfire, 0)

        @pl.when(s == NS - 1)
        def _():
            pltpu.async_copy(bufs[0].at[pl.ds(0, 16)],
                             agg_sh.at[pl.ds(NS * ZSTRIPE, LAST_EXTRA)], zsem)

        def zdrain(i, carry):
            pltpu.make_async_copy(bufs[0].at[pl.ds(0, 16)],
                                  agg_sh.at[pl.ds(0, 16)], zsem).wait()
            return carry
        lax.fori_loop(0, ZSTRIPE // 16, zdrain, 0)

        @pl.when(s == NS - 1)
        def _():
            pltpu.make_async_copy(bufs[0].at[pl.ds(0, 16)],
                                  agg_sh.at[pl.ds(0, LAST_EXTRA)], zsem).wait()

        for q in range(NI):
            pltpu.async_copy(eidx_hbm.at[start + q],
                             ibuf.at[pl.ds(2 * q, 2)], isems[q])
        for b in range(NB):
            pltpu.make_async_copy(
                eidx_hbm.at[start + b],
                ibuf.at[pl.ds(2 * b, 2)], isems[b]).wait()
            pltpu.async_copy(feat_hbm.at[ibuf.at[2 * b]], bufs[b], gsems[b])

        plsc.subcore_barrier()

        cnt = jnp.where(extra, BASE_CHUNKS + 1, BASE_CHUNKS)

        def group_body(g, carry):
            base = g * NI
            for h in range(NI // NB):
                for b in range(NB):
                    q = h * NB + b
                    j = base + q
                    pltpu.make_async_copy(
                        feat_hbm.at[ibuf.at[2 * q]], bufs[b], gsems[b]).wait()
                    pltpu.async_copy(
                        bufs[b], agg_sh.at[ibuf.at[2 * q + 1]], ssems[b],
                        add=True)
                for b in range(NB):
                    q = h * NB + b
                    j = base + q
                    pltpu.make_async_copy(
                        bufs[b], agg_sh.at[ibuf.at[2 * q + 1]], ssems[b]).wait()

                    @pl.when(j + NI < cnt)
                    def _():
                        pltpu.async_copy(
                            eidx_hbm.at[start + j + NI],
                            ibuf.at[pl.ds(2 * q, 2)], isems[q])

                    @pl.when(j + NB < cnt)
                    def _():
                        qn = (q + NB) % NI
                        pltpu.make_async_copy(
                            eidx_hbm.at[start + j + NB],
                            ibuf.at[pl.ds(2 * qn, 2)], isems[qn]).wait()
                        pltpu.async_copy(
                            feat_hbm.at[ibuf.at[2 * qn]], bufs[b], gsems[b])
            return carry
        lax.fori_loop(0, GROUPS, group_body, 0)

        @pl.when(extra)
        def _():
            pltpu.make_async_copy(
                feat_hbm.at[ibuf.at[0]], bufs[0], gsems[0]).wait()
            pltpu.async_copy(
                bufs[0], agg_sh.at[ibuf.at[1]], ssems[0], add=True)
            pltpu.make_async_copy(
                bufs[0], agg_sh.at[ibuf.at[1]], ssems[0]).wait()

        plsc.subcore_barrier()

        pltpu.sync_copy(
            agg_sh.at[pl.ds(s * ZSTRIPE, ZSTRIPE)],
            out_hbm.at[c, pl.ds(s * ZSTRIPE, ZSTRIPE)])

        @pl.when(s == NS - 1)
        def _():
            pltpu.sync_copy(
                agg_sh.at[pl.ds(NS * ZSTRIPE, LAST_EXTRA)],
                out_hbm.at[c, pl.ds(NS * ZSTRIPE, LAST_EXTRA)])

    return agg_kernel(features, eidx)


def _tc_update(partials, W, b2d):
    BLK = 1000

    def body(p_ref, w_ref, b_ref, o_ref):
        acc = p_ref[0] + p_ref[1]
        h = jnp.dot(acc, w_ref[...], preferred_element_type=jnp.float32)
        o_ref[...] = jnp.maximum(h + b_ref[...], 0.0)

    return pl.pallas_call(
        body,
        grid=(N_NODES_K // BLK,),
        in_specs=[
            pl.BlockSpec((NC, BLK, D), lambda i: (0, i, 0)),
            pl.BlockSpec((D, D), lambda i: (0, 0)),
            pl.BlockSpec((1, D), lambda i: (0, 0)),
        ],
        out_specs=pl.BlockSpec((BLK, D), lambda i: (i, 0)),
        out_shape=jax.ShapeDtypeStruct((N_NODES_K, D), jnp.float32),
    )(partials, W, b2d)


def kernel(features, edge_index, W, b):
    src = edge_index[0].astype(jnp.int32)
    dst = edge_index[1].astype(jnp.int32)
    eidx = jnp.stack(
        [src.reshape(N_CHUNKS, CHUNK), dst.reshape(N_CHUNKS, CHUNK)], axis=1)

    partials = _sc_aggregate(features, eidx)
    return _tc_update(partials, W, b.reshape(1, D))


# speedup vs baseline: 1.0939x; 1.0939x over previous
import functools

import jax
import jax.numpy as jnp
from jax import lax
from jax.experimental import pallas as pl
from jax.experimental.pallas import tpu as pltpu
from jax.experimental.pallas import tpu_sc as plsc

N_NODES_K = 10000
D = 128
N_EDGES_K = 320000

NC = 2
NS = 16
NW = NC * NS

CHUNK = 128
N_CHUNKS = N_EDGES_K // CHUNK
BASE_CHUNKS = N_CHUNKS // NW
EXTRA_TILES = N_CHUNKS - BASE_CHUNKS * NW

NB = 3
NI = 2 * NB
GROUPS = BASE_CHUNKS // NI

ZSTRIPE = N_NODES_K // NS // 16 * 16
LAST_EXTRA = N_NODES_K - NS * ZSTRIPE


def _sc_aggregate(features, e1d):
    mesh = plsc.VectorSubcoreMesh(core_axis_name="c", subcore_axis_name="s")

    @functools.partial(
        pl.kernel,
        out_type=jax.ShapeDtypeStruct((NC, N_NODES_K, D), jnp.float32),
        mesh=mesh,
        scratch_types=(
            [pltpu.VMEM((CHUNK, D), jnp.float32)] * NB
            + [pltpu.VMEM((2, CHUNK), jnp.int32)] * NI
            + [pltpu.VMEM_SHARED((N_NODES_K, D), jnp.float32)]
            + [pltpu.SemaphoreType.DMA] * (2 * NB + NI + 1)
        ),
    )
    def agg_kernel(feat_hbm, eidx_hbm, out_hbm, *rest):
        bufs = rest[:NB]
        ibufs = rest[NB:NB + NI]
        agg_sh = rest[NB + NI]
        sems = rest[NB + NI + 1:]
        gsems = sems[:NB]
        ssems = sems[NB:2 * NB]
        isems = sems[2 * NB:2 * NB + NI]
        zsem = sems[2 * NB + NI]

        c = lax.axis_index("c")
        s = lax.axis_index("s")
        wid = c * NS + s
        start = wid * BASE_CHUNKS + jnp.minimum(wid, EXTRA_TILES)
        extra = wid < EXTRA_TILES

        zvec = jnp.zeros((16,), jnp.float32)
        for r in range(16):
            for col in range(D // 16):
                bufs[0][r, pl.ds(col * 16, 16)] = zvec

        def zfire(i, carry):
            pltpu.async_copy(bufs[0].at[pl.ds(0, 16)],
                             agg_sh.at[pl.ds(s * ZSTRIPE + i * 16, 16)], zsem)
            return carry
        lax.fori_loop(0, ZSTRIPE // 16, zfire, 0)

        @pl.when(s == NS - 1)
        def _():
            pltpu.async_copy(bufs[0].at[pl.ds(0, 16)],
                             agg_sh.at[pl.ds(NS * ZSTRIPE, LAST_EXTRA)], zsem)

        def zdrain(i, carry):
            pltpu.make_async_copy(bufs[0].at[pl.ds(0, 16)],
                                  agg_sh.at[pl.ds(0, 16)], zsem).wait()
            return carry
        lax.fori_loop(0, ZSTRIPE // 16, zdrain, 0)

        @pl.when(s == NS - 1)
        def _():
            pltpu.make_async_copy(bufs[0].at[pl.ds(0, 16)],
                                  agg_sh.at[pl.ds(0, LAST_EXTRA)], zsem).wait()

        def fire_idx(chunk, slot):
            off = (start + chunk) * CHUNK
            pltpu.async_copy(eidx_hbm.at[pl.ds(off, CHUNK)],
                             ibufs[slot].at[0], isems[slot])
            pltpu.async_copy(eidx_hbm.at[pl.ds(N_EDGES_K + off, CHUNK)],
                             ibufs[slot].at[1], isems[slot])

        def wait_idx(slot):
            for half in range(2):
                pltpu.make_async_copy(
                    eidx_hbm.at[pl.ds(0, CHUNK)],
                    ibufs[slot].at[half], isems[slot]).wait()

        for q in range(NI):
            fire_idx(q, q)
        for b in range(NB):
            wait_idx(b)
            pltpu.async_copy(feat_hbm.at[ibufs[b].at[0]], bufs[b], gsems[b])

        plsc.subcore_barrier()

        cnt = jnp.where(extra, BASE_CHUNKS + 1, BASE_CHUNKS)

        def group_body(g, carry):
            base = g * NI
            for h in range(2):
                for b in range(NB):
                    q = h * NB + b
                    j = base + q
                    pltpu.make_async_copy(
                        feat_hbm.at[ibufs[q].at[0]], bufs[b], gsems[b]).wait()
                    pltpu.async_copy(
                        bufs[b], agg_sh.at[ibufs[q].at[1]], ssems[b], add=True)
                for b in range(NB):
                    q = h * NB + b
                    j = base + q
                    pltpu.make_async_copy(
                        bufs[b], agg_sh.at[ibufs[q].at[1]], ssems[b]).wait()

                    @pl.when(j + NI < cnt)
                    def _():
                        fire_idx(j + NI, q)

                    @pl.when(j + NB < cnt)
                    def _():
                        qn = (q + NB) % NI
                        wait_idx(qn)
                        pltpu.async_copy(
                            feat_hbm.at[ibufs[qn].at[0]], bufs[b], gsems[b])
            return carry
        lax.fori_loop(0, GROUPS, group_body, 0)

        @pl.when(extra)
        def _():
            pltpu.make_async_copy(
                feat_hbm.at[ibufs[0].at[0]], bufs[0], gsems[0]).wait()
            pltpu.async_copy(
                bufs[0], agg_sh.at[ibufs[0].at[1]], ssems[0], add=True)
            pltpu.make_async_copy(
                bufs[0], agg_sh.at[ibufs[0].at[1]], ssems[0]).wait()

        plsc.subcore_barrier()

        pltpu.sync_copy(
            agg_sh.at[pl.ds(s * ZSTRIPE, ZSTRIPE)],
            out_hbm.at[c, pl.ds(s * ZSTRIPE, ZSTRIPE)])

        @pl.when(s == NS - 1)
        def _():
            pltpu.sync_copy(
                agg_sh.at[pl.ds(NS * ZSTRIPE, LAST_EXTRA)],
                out_hbm.at[c, pl.ds(NS * ZSTRIPE, LAST_EXTRA)])

    return agg_kernel(features, e1d)


def _tc_update(partials, W, b2d):
    BLK = 1000

    def body(p_ref, w_ref, b_ref, o_ref):
        acc = p_ref[0] + p_ref[1]
        h = jnp.dot(acc, w_ref[...], preferred_element_type=jnp.float32)
        o_ref[...] = jnp.maximum(h + b_ref[...], 0.0)

    return pl.pallas_call(
        body,
        grid=(N_NODES_K // BLK,),
        in_specs=[
            pl.BlockSpec((NC, BLK, D), lambda i: (0, i, 0)),
            pl.BlockSpec((D, D), lambda i: (0, 0)),
            pl.BlockSpec((1, D), lambda i: (0, 0)),
        ],
        out_specs=pl.BlockSpec((BLK, D), lambda i: (i, 0)),
        out_shape=jax.ShapeDtypeStruct((N_NODES_K, D), jnp.float32),
    )(partials, W, b2d)


def kernel(features, edge_index, W, b):
    e1d = edge_index.astype(jnp.int32).reshape(2 * N_EDGES_K)

    partials = _sc_aggregate(features, e1d)
    return _tc_update(partials, W, b.reshape(1, D))


# speedup vs baseline: 1.1090x; 1.0138x over previous
import functools

import jax
import jax.numpy as jnp
from jax import lax
from jax.experimental import pallas as pl
from jax.experimental.pallas import tpu as pltpu
from jax.experimental.pallas import tpu_sc as plsc

N_NODES_K = 10000
D = 128
N_EDGES_K = 320000

NC = 2
NS = 16
NW = NC * NS

CHUNK = 128
N_CHUNKS = N_EDGES_K // CHUNK
BASE_CHUNKS = N_CHUNKS // NW
EXTRA_TILES = N_CHUNKS - BASE_CHUNKS * NW

NB = 3
NI = 2 * NB
GROUPS = BASE_CHUNKS // NI

ZSTRIPE = N_NODES_K // NS // 16 * 16
LAST_EXTRA = N_NODES_K - NS * ZSTRIPE


def _sc_aggregate(features, e1d):
    mesh = plsc.VectorSubcoreMesh(core_axis_name="c", subcore_axis_name="s")

    @functools.partial(
        pl.kernel,
        out_type=jax.ShapeDtypeStruct((NC, N_NODES_K, D), jnp.float32),
        mesh=mesh,
        scratch_types=(
            [pltpu.VMEM((CHUNK, D), jnp.float32)] * NB
            + [pltpu.VMEM((2, CHUNK), jnp.int32)] * NI
            + [pltpu.VMEM_SHARED((N_NODES_K, D), jnp.float32)]
            + [pltpu.SemaphoreType.DMA] * (2 * NB + NI + 1)
        ),
    )
    def agg_kernel(feat_hbm, eidx_hbm, out_hbm, *rest):
        bufs = rest[:NB]
        ibufs = rest[NB:NB + NI]
        agg_sh = rest[NB + NI]
        sems = rest[NB + NI + 1:]
        gsems = sems[:NB]
        ssems = sems[NB:2 * NB]
        isems = sems[2 * NB:2 * NB + NI]
        zsem = sems[2 * NB + NI]

        c = lax.axis_index("c")
        s = lax.axis_index("s")
        wid = c * NS + s
        start = wid * BASE_CHUNKS + jnp.minimum(wid, EXTRA_TILES)
        extra = wid < EXTRA_TILES

        zvec = jnp.zeros((16,), jnp.float32)
        for r in range(16):
            for col in range(D // 16):
                bufs[0][r, pl.ds(col * 16, 16)] = zvec

        def zfire(i, carry):
            pltpu.async_copy(bufs[0].at[pl.ds(0, 16)],
                             agg_sh.at[pl.ds(s * ZSTRIPE + i * 16, 16)], zsem)
            return carry
        lax.fori_loop(0, ZSTRIPE // 16, zfire, 0)

        @pl.when(s == NS - 1)
        def _():
            pltpu.async_copy(bufs[0].at[pl.ds(0, 16)],
                             agg_sh.at[pl.ds(NS * ZSTRIPE, LAST_EXTRA)], zsem)

        def zdrain(i, carry):
            pltpu.make_async_copy(bufs[0].at[pl.ds(0, 16)],
                                  agg_sh.at[pl.ds(0, 16)], zsem).wait()
            return carry
        lax.fori_loop(0, ZSTRIPE // 16, zdrain, 0)

        @pl.when(s == NS - 1)
        def _():
            pltpu.make_async_copy(bufs[0].at[pl.ds(0, 16)],
                                  agg_sh.at[pl.ds(0, LAST_EXTRA)], zsem).wait()

        def fire_idx(chunk, slot):
            off = (start + chunk) * CHUNK
            pltpu.async_copy(eidx_hbm.at[0, pl.ds(off, CHUNK)],
                             ibufs[slot].at[0], isems[slot])
            pltpu.async_copy(eidx_hbm.at[1, pl.ds(off, CHUNK)],
                             ibufs[slot].at[1], isems[slot])

        def wait_idx(slot):
            for half in range(2):
                pltpu.make_async_copy(
                    eidx_hbm.at[0, pl.ds(0, CHUNK)],
                    ibufs[slot].at[half], isems[slot]).wait()

        for q in range(NI):
            fire_idx(q, q)
        for b in range(NB):
            wait_idx(b)
            pltpu.async_copy(feat_hbm.at[ibufs[b].at[0]], bufs[b], gsems[b])

        plsc.subcore_barrier()

        cnt = jnp.where(extra, BASE_CHUNKS + 1, BASE_CHUNKS)

        def group_body(g, carry):
            base = g * NI
            for h in range(2):
                for b in range(NB):
                    q = h * NB + b
                    j = base + q
                    pltpu.make_async_copy(
                        feat_hbm.at[ibufs[q].at[0]], bufs[b], gsems[b]).wait()
                    pltpu.async_copy(
                        bufs[b], agg_sh.at[ibufs[q].at[1]], ssems[b], add=True)
                for b in range(NB):
                    q = h * NB + b
                    j = base + q
                    pltpu.make_async_copy(
                        bufs[b], agg_sh.at[ibufs[q].at[1]], ssems[b]).wait()

                    @pl.when(j + NI < cnt)
                    def _():
                        fire_idx(j + NI, q)

                    @pl.when(j + NB < cnt)
                    def _():
                        qn = (q + NB) % NI
                        wait_idx(qn)
                        pltpu.async_copy(
                            feat_hbm.at[ibufs[qn].at[0]], bufs[b], gsems[b])
            return carry
        lax.fori_loop(0, GROUPS, group_body, 0)

        @pl.when(extra)
        def _():
            pltpu.make_async_copy(
                feat_hbm.at[ibufs[0].at[0]], bufs[0], gsems[0]).wait()
            pltpu.async_copy(
                bufs[0], agg_sh.at[ibufs[0].at[1]], ssems[0], add=True)
            pltpu.make_async_copy(
                bufs[0], agg_sh.at[ibufs[0].at[1]], ssems[0]).wait()

        plsc.subcore_barrier()

        pltpu.sync_copy(
            agg_sh.at[pl.ds(s * ZSTRIPE, ZSTRIPE)],
            out_hbm.at[c, pl.ds(s * ZSTRIPE, ZSTRIPE)])

        @pl.when(s == NS - 1)
        def _():
            pltpu.sync_copy(
                agg_sh.at[pl.ds(NS * ZSTRIPE, LAST_EXTRA)],
                out_hbm.at[c, pl.ds(NS * ZSTRIPE, LAST_EXTRA)])

    return agg_kernel(features, e1d)


def _tc_update(partials, W, b2d):
    BLK = 1000

    def body(p_ref, w_ref, b_ref, o_ref):
        acc = p_ref[0] + p_ref[1]
        h = jnp.dot(acc, w_ref[...], preferred_element_type=jnp.float32)
        o_ref[...] = jnp.maximum(h + b_ref[...], 0.0)

    return pl.pallas_call(
        body,
        grid=(N_NODES_K // BLK,),
        in_specs=[
            pl.BlockSpec((NC, BLK, D), lambda i: (0, i, 0)),
            pl.BlockSpec((D, D), lambda i: (0, 0)),
            pl.BlockSpec((1, D), lambda i: (0, 0)),
        ],
        out_specs=pl.BlockSpec((BLK, D), lambda i: (i, 0)),
        out_shape=jax.ShapeDtypeStruct((N_NODES_K, D), jnp.float32),
    )(partials, W, b2d)


def kernel(features, edge_index, W, b):
    partials = _sc_aggregate(features, edge_index.astype(jnp.int32))
    return _tc_update(partials, W, b.reshape(1, D))


# speedup vs baseline: 1.1352x; 1.0236x over previous
import functools

import jax
import jax.numpy as jnp
from jax import lax
from jax.experimental import pallas as pl
from jax.experimental.pallas import tpu as pltpu
from jax.experimental.pallas import tpu_sc as plsc

N_NODES_K = 10000
D = 128
N_EDGES_K = 320000

NC = 2
NS = 16
NW = NC * NS

CHUNK = 128
N_CHUNKS = N_EDGES_K // CHUNK
BASE_CHUNKS = N_CHUNKS // NW
EXTRA_TILES = N_CHUNKS - BASE_CHUNKS * NW

NB = 3
NI = 2 * NB
GROUPS = BASE_CHUNKS // NI

ZSTRIPE = N_NODES_K // NS // 16 * 16
LAST_EXTRA = N_NODES_K - NS * ZSTRIPE


def _sc_aggregate(features, e1d):
    mesh = plsc.VectorSubcoreMesh(core_axis_name="c", subcore_axis_name="s")

    @functools.partial(
        pl.kernel,
        out_type=jax.ShapeDtypeStruct((NC, N_NODES_K, D), jnp.float32),
        mesh=mesh,
        scratch_types=(
            [pltpu.VMEM((CHUNK, D), jnp.float32)] * NB
            + [pltpu.VMEM((2, CHUNK), jnp.int32)] * NI
            + [pltpu.VMEM_SHARED((N_NODES_K, D), jnp.float32)]
            + [pltpu.SemaphoreType.DMA] * (2 * NB + NI + 1)
        ),
    )
    def agg_kernel(feat_hbm, eidx_hbm, out_hbm, *rest):
        bufs = rest[:NB]
        ibufs = rest[NB:NB + NI]
        agg_sh = rest[NB + NI]
        sems = rest[NB + NI + 1:]
        gsems = sems[:NB]
        ssems = sems[NB:2 * NB]
        isems = sems[2 * NB:2 * NB + NI]
        zsem = sems[2 * NB + NI]

        c = lax.axis_index("c")
        s = lax.axis_index("s")
        wid = c * NS + s
        start = wid * BASE_CHUNKS + jnp.minimum(wid, EXTRA_TILES)
        extra = wid < EXTRA_TILES

        zvec = jnp.zeros((16,), jnp.float32)
        for r in range(16):
            for col in range(D // 16):
                bufs[0][r, pl.ds(col * 16, 16)] = zvec

        def zfire(i, carry):
            pltpu.async_copy(bufs[0].at[pl.ds(0, 16)],
                             agg_sh.at[pl.ds(s * ZSTRIPE + i * 16, 16)], zsem)
            return carry
        lax.fori_loop(0, ZSTRIPE // 16, zfire, 0)

        @pl.when(s == NS - 1)
        def _():
            pltpu.async_copy(bufs[0].at[pl.ds(0, 16)],
                             agg_sh.at[pl.ds(NS * ZSTRIPE, LAST_EXTRA)], zsem)

        def zdrain(i, carry):
            pltpu.make_async_copy(bufs[0].at[pl.ds(0, 16)],
                                  agg_sh.at[pl.ds(0, 16)], zsem).wait()
            return carry
        lax.fori_loop(0, ZSTRIPE // 16, zdrain, 0)

        @pl.when(s == NS - 1)
        def _():
            pltpu.make_async_copy(bufs[0].at[pl.ds(0, 16)],
                                  agg_sh.at[pl.ds(0, LAST_EXTRA)], zsem).wait()

        def fire_idx(chunk, slot):
            off = (start + chunk) * CHUNK
            pltpu.async_copy(eidx_hbm.at[0, pl.ds(off, CHUNK)],
                             ibufs[slot].at[0], isems[slot])
            pltpu.async_copy(eidx_hbm.at[1, pl.ds(off, CHUNK)],
                             ibufs[slot].at[1], isems[slot])

        def wait_idx(slot):
            for half in range(2):
                pltpu.make_async_copy(
                    eidx_hbm.at[0, pl.ds(0, CHUNK)],
                    ibufs[slot].at[half], isems[slot]).wait()

        for q in range(NI):
            fire_idx(q, q)
        for b in range(NB):
            wait_idx(b)
            pltpu.async_copy(feat_hbm.at[ibufs[b].at[0]], bufs[b], gsems[b])

        plsc.subcore_barrier()

        cnt = jnp.where(extra, BASE_CHUNKS + 1, BASE_CHUNKS)

        def group_body(g, carry):
            base = g * NI
            for h in range(2):
                for b in range(NB):
                    q = h * NB + b
                    j = base + q
                    pltpu.make_async_copy(
                        feat_hbm.at[ibufs[q].at[0]], bufs[b], gsems[b]).wait()
                    pltpu.async_copy(
                        bufs[b], agg_sh.at[ibufs[q].at[1]], ssems[b], add=True)
                for b in range(NB):
                    q = h * NB + b
                    j = base + q
                    pltpu.make_async_copy(
                        bufs[b], agg_sh.at[ibufs[q].at[1]], ssems[b]).wait()

                    @pl.when(j + NI < cnt)
                    def _():
                        fire_idx(j + NI, q)

                    @pl.when(j + NB < cnt)
                    def _():
                        qn = (q + NB) % NI
                        wait_idx(qn)
                        pltpu.async_copy(
                            feat_hbm.at[ibufs[qn].at[0]], bufs[b], gsems[b])
            return carry
        lax.fori_loop(0, GROUPS, group_body, 0)

        @pl.when(extra)
        def _():
            pltpu.make_async_copy(
                feat_hbm.at[ibufs[0].at[0]], bufs[0], gsems[0]).wait()
            pltpu.async_copy(
                bufs[0], agg_sh.at[ibufs[0].at[1]], ssems[0], add=True)
            pltpu.make_async_copy(
                bufs[0], agg_sh.at[ibufs[0].at[1]], ssems[0]).wait()

        plsc.subcore_barrier()

        pltpu.sync_copy(
            agg_sh.at[pl.ds(s * ZSTRIPE, ZSTRIPE)],
            out_hbm.at[c, pl.ds(s * ZSTRIPE, ZSTRIPE)])

        @pl.when(s == NS - 1)
        def _():
            pltpu.sync_copy(
                agg_sh.at[pl.ds(NS * ZSTRIPE, LAST_EXTRA)],
                out_hbm.at[c, pl.ds(NS * ZSTRIPE, LAST_EXTRA)])

    return agg_kernel(features, e1d)


def _tc_update(partials, W, b2d):
    BLK = 2000

    def body(p_ref, w_ref, b_ref, o_ref):
        acc = p_ref[0] + p_ref[1]
        h = jnp.dot(acc, w_ref[...], preferred_element_type=jnp.float32)
        o_ref[...] = jnp.maximum(h + b_ref[...], 0.0)

    return pl.pallas_call(
        body,
        grid=(N_NODES_K // BLK,),
        in_specs=[
            pl.BlockSpec((NC, BLK, D), lambda i: (0, i, 0)),
            pl.BlockSpec((D, D), lambda i: (0, 0)),
            pl.BlockSpec((1, D), lambda i: (0, 0)),
        ],
        out_specs=pl.BlockSpec((BLK, D), lambda i: (i, 0)),
        out_shape=jax.ShapeDtypeStruct((N_NODES_K, D), jnp.float32),
    )(partials, W, b2d)


def kernel(features, edge_index, W, b):
    partials = _sc_aggregate(features, edge_index.astype(jnp.int32))
    return _tc_update(partials, W, b.reshape(1, D))
